# baseline (speedup 1.0000x reference)
import functools

import jax
import jax.numpy as jnp
from jax import lax
from jax.experimental import pallas as pl
from jax.experimental.pallas import tpu as pltpu
from jax.experimental.pallas import tpu_sc as plsc

NC = 2
NS = 16
NW = NC * NS
CH = 128
NB = 4
NB2 = 2
LANES = 16
SPLIT_SCAT = 0.28
SPLIT_DOT = 0.22

_MESH = plsc.VectorSubcoreMesh(
    core_axis_name="c", subcore_axis_name="s", num_cores=NC, num_subcores=NS
)


def _core_split(ncht, f0, align):
    per_core_units = ncht // (16 * align)
    u0 = min(max(int(round(f0 * per_core_units)), 1), per_core_units - 1)
    return u0 * align, (per_core_units - u0) * align


def _sc_deg(dstp2, ones, zrow, n_acc, ncht):
    nch = ncht // NW
    stripe = n_acc // NS

    def body(dstp_hbm, ones_hbm, z_hbm, out_hbm, didx, ones_v, stage_v, deg_sh, sem):
        cid = lax.axis_index("c")
        sid = lax.axis_index("s")
        wid = sid * NC + cid
        pltpu.sync_copy(z_hbm, stage_v)
        pltpu.sync_copy(stage_v, deg_sh.at[pl.ds(sid * stripe, stripe)])
        pltpu.sync_copy(dstp_hbm.at[pl.ds(wid * nch, nch)], didx)
        pltpu.sync_copy(ones_hbm, ones_v)
        plsc.subcore_barrier()

        def fire(i, c):
            pltpu.async_copy(ones_v, deg_sh.at[didx.at[i]], sem, add=True)
            return c

        lax.fori_loop(0, nch, fire, 0)

        def drain(i, c):
            pltpu.make_async_copy(ones_v, deg_sh.at[didx.at[i]], sem).wait()
            return c

        lax.fori_loop(0, nch, drain, 0)
        plsc.subcore_barrier()
        pltpu.sync_copy(deg_sh.at[pl.ds(sid * stripe, stripe)], stage_v)
        pltpu.sync_copy(
            stage_v,
            out_hbm.at[pl.ds(cid * n_acc + sid * stripe, stripe)],
        )

    return pl.kernel(
        body,
        out_type=jax.ShapeDtypeStruct((NC * n_acc,), jnp.float32),
        mesh=_MESH,
        scratch_types=[
            pltpu.VMEM((nch, CH), jnp.int32),
            pltpu.VMEM((CH,), jnp.float32),
            pltpu.VMEM((stripe,), jnp.float32),
            pltpu.VMEM_SHARED((n_acc,), jnp.float32),
            pltpu.SemaphoreType.DMA,
        ],
    )(dstp2, ones, zrow)


@functools.lru_cache(maxsize=None)
def _sc_scatter_kernel(n, d, n_acc, ncht, nch0, nch1):
    SB = 8
    stripe = n_acc // NS

    def body(g_hbm, srcp_hbm, dstp_hbm, z_hbm, out_hbm, sidx, didx, rows, acc_sh, *sems):
        gsems, ssems, isems_s, isems_d = sems[:2], sems[2:4], sems[4:6], sems[6:8]
        cid = lax.axis_index("c")
        sid = lax.axis_index("s")
        nch_w = jnp.where(cid == 0, nch0, nch1)
        cb = pl.multiple_of(jnp.where(cid == 0, sid * nch0, 16 * nch0 + sid * nch1), 8)
        nsc_w = nch_w // SB
        npair_w = nsc_w // 2
        pltpu.sync_copy(z_hbm, acc_sh.at[pl.ds(sid * stripe, stripe)])
        for sl in range(2):
            pltpu.async_copy(srcp_hbm.at[pl.ds(cb + sl * SB, SB)], sidx.at[sl], isems_s[sl])
            pltpu.async_copy(dstp_hbm.at[pl.ds(cb + sl * SB, SB)], didx.at[sl], isems_d[sl])
        pltpu.make_async_copy(srcp_hbm.at[pl.ds(cb, SB)], sidx.at[0], isems_s[0]).wait()
        pltpu.make_async_copy(dstp_hbm.at[pl.ds(cb, SB)], didx.at[0], isems_d[0]).wait()
        pltpu.async_copy(g_hbm.at[sidx.at[0, 0]], rows.at[pl.ds(0, CH)], gsems[0])
        plsc.subcore_barrier()

        def chunk(s0, ph, b, first):
            rs = b % 2
            cur = rows.at[pl.ds(rs * CH, CH)]
            nxt = rows.at[pl.ds((1 - rs) * CH, CH)]
            pltpu.make_async_copy(g_hbm.at[sidx.at[ph, b]], cur, gsems[rs]).wait()
            pltpu.async_copy(cur, acc_sh.at[didx.at[ph, b]], ssems[rs], add=True)
            if not first:
                pidx = didx.at[ph, b - 1] if b > 0 else didx.at[1 - ph, SB - 1]
                pltpu.make_async_copy(nxt, acc_sh.at[pidx], ssems[1 - rs]).wait()
                if b == 0:
                    sr = cb + lax.rem(s0 + 1, nsc_w) * SB
                    pltpu.async_copy(
                        srcp_hbm.at[pl.ds(sr, SB)], sidx.at[1 - ph], isems_s[1 - ph])
                    pltpu.async_copy(
                        dstp_hbm.at[pl.ds(sr, SB)], didx.at[1 - ph], isems_d[1 - ph])
            if b == SB - 1:
                srn = cb + lax.rem(s0 + 1, nsc_w) * SB
                pltpu.make_async_copy(
                    srcp_hbm.at[pl.ds(srn, SB)], sidx.at[1 - ph], isems_s[1 - ph]).wait()
                pltpu.make_async_copy(
                    dstp_hbm.at[pl.ds(srn, SB)], didx.at[1 - ph], isems_d[1 - ph]).wait()
                nref = sidx.at[1 - ph, 0]
            else:
                nref = sidx.at[ph, b + 1]
            pltpu.async_copy(g_hbm.at[nref], nxt, gsems[1 - rs])

        for ph in range(2):
            for b in range(SB):
                chunk(ph, ph, b, ph == 0 and b == 0)

        def pair(p, carry):
            for ph in range(2):
                for b in range(SB):
                    chunk(p * 2 + ph, ph, b, False)
            return carry

        lax.fori_loop(1, npair_w, pair, 0)
        pltpu.make_async_copy(
            rows.at[pl.ds(CH, CH)], acc_sh.at[didx.at[1, SB - 1]], ssems[1]).wait()
        pltpu.make_async_copy(
            g_hbm.at[sidx.at[0, 0]], rows.at[pl.ds(0, CH)], gsems[0]).wait()
        plsc.subcore_barrier()
        pltpu.sync_copy(
            acc_sh.at[pl.ds(sid * stripe, stripe)],
            out_hbm.at[cid, pl.ds(sid * stripe, stripe)],
        )

    return pl.kernel(
        body,
        out_type=jax.ShapeDtypeStruct((NC, n_acc, d), jnp.float32),
        mesh=_MESH,
        scratch_types=[
            pltpu.VMEM((2, SB, CH), jnp.int32),
            pltpu.VMEM((2, SB, CH), jnp.int32),
            pltpu.VMEM((2 * CH, d), jnp.float32),
            pltpu.VMEM_SHARED((n_acc, d), jnp.float32),
        ] + [pltpu.SemaphoreType.DMA] * 8,
    )


def _sc_scatter(g, srcp2, dstp2, zblk, n_acc, ncht, split):
    n, d = g.shape
    return _sc_scatter_kernel(n, d, n_acc, ncht, *split)(g, srcp2, dstp2, zblk)


def _sc_edge_dot(h2, srcp2, dstp2, ncht, e_pad, split):
    n, d = h2.shape
    nch0, nch1 = split
    nmax = max(nch0, nch1)
    nv = d // LANES

    def body(h_hbm, s_hbm, t_hbm, out_hbm, sidx, didx, rs, rd, dots, *sems):
        gs, gd, osems = sems[:NB2], sems[NB2:2 * NB2], sems[2 * NB2:]
        cid = lax.axis_index("c")
        sid = lax.axis_index("s")
        nch_w = jnp.where(cid == 0, nch0, nch1)
        cb = pl.multiple_of(jnp.where(cid == 0, sid * nch0, 16 * nch0 + sid * nch1), 8)
        nblk_w = nch_w // NB2
        obase = cb * CH
        pltpu.sync_copy(s_hbm.at[pl.ds(cb, nmax)], sidx)
        pltpu.sync_copy(t_hbm.at[pl.ds(cb, nmax)], didx)
        for b in range(NB2):
            pltpu.async_copy(h_hbm.at[sidx.at[b]], rs.at[pl.ds(b * CH, CH)], gs[b])
            pltpu.async_copy(h_hbm.at[didx.at[b]], rd.at[pl.ds(b * CH, CH)], gd[b])
        lane = lax.iota(jnp.int32, LANES)

        def chunk(i, b, first):
            bufs = rs.at[pl.ds(b * CH, CH)]
            bufd = rd.at[pl.ds(b * CH, CH)]
            pltpu.make_async_copy(h_hbm.at[sidx.at[i]], bufs, gs[b]).wait()
            pltpu.make_async_copy(h_hbm.at[didx.at[i]], bufd, gd[b]).wait()
            if not first:
                pltpu.make_async_copy(
                    dots.at[pl.ds(b * CH, CH)],
                    out_hbm.at[pl.ds(obase + (i - NB2) * CH, CH)],
                    osems[b],
                ).wait()
            for gi in range(CH // LANES):
                def edot(k, vec):
                    e = b * CH + gi * LANES + k
                    v = rs[e, pl.ds(0, LANES)] * rd[e, pl.ds(0, LANES)]
                    for jj in range(1, nv):
                        v = v + rs[e, pl.ds(jj * LANES, LANES)] * rd[e, pl.ds(jj * LANES, LANES)]
                    for sh in (8, 4, 2, 1):
                        v = v + jnp.take_along_axis(
                            v, lane ^ sh, axis=0, mode="promise_in_bounds")
                    return jnp.where(lane == k, v, vec)

                dots[pl.ds(b * CH + gi * LANES, LANES)] = lax.fori_loop(
                    0, LANES, edot, jnp.zeros((LANES,), jnp.float32))
            pltpu.async_copy(
                dots.at[pl.ds(b * CH, CH)],
                out_hbm.at[pl.ds(obase + i * CH, CH)],
                osems[b],
            )
            j = lax.rem(i + NB2, nch_w)
            pltpu.async_copy(h_hbm.at[sidx.at[j]], bufs, gs[b])
            pltpu.async_copy(h_hbm.at[didx.at[j]], bufd, gd[b])

        for b in range(NB2):
            chunk(b, b, True)

        def block(i0, carry):
            for b in range(NB2):
                chunk(i0 * NB2 + b, b, False)
            return carry

        lax.fori_loop(1, nblk_w, block, 0)
        for b in range(NB2):
            i = nch_w - NB2 + b
            pltpu.make_async_copy(
                dots.at[pl.ds(b * CH, CH)],
                out_hbm.at[pl.ds(obase + i * CH, CH)],
                osems[b],
            ).wait()
            pltpu.make_async_copy(h_hbm.at[sidx.at[b]], rs.at[pl.ds(b * CH, CH)], gs[b]).wait()
            pltpu.make_async_copy(h_hbm.at[didx.at[b]], rd.at[pl.ds(b * CH, CH)], gd[b]).wait()

    return pl.kernel(
        body,
        out_type=jax.ShapeDtypeStruct((e_pad,), jnp.float32),
        mesh=_MESH,
        scratch_types=[
            pltpu.VMEM((nmax, CH), jnp.int32),
            pltpu.VMEM((nmax, CH), jnp.int32),
            pltpu.VMEM((NB2 * CH, d), jnp.float32),
            pltpu.VMEM((NB2 * CH, d), jnp.float32),
            pltpu.VMEM((NB2 * CH,), jnp.float32),
        ] + [pltpu.SemaphoreType.DMA] * (3 * NB2),
    )(h2, srcp2, dstp2)


def _tc_layer1(deg3, x, w1, n):
    d = x.shape[1]

    def body(deg_ref, x_ref, w_ref, dinv_ref, g_ref):
        dv = lax.rsqrt(deg_ref[0] + deg_ref[1] + 1.0)[:n]
        h = jnp.dot(x_ref[...], w_ref[...], preferred_element_type=jnp.float32,
                    precision=lax.Precision.HIGHEST)
        dinv_ref[...] = dv
        g_ref[...] = h * dv

    return pl.pallas_call(
        body,
        out_shape=(
            jax.ShapeDtypeStruct((n, 1), jnp.float32),
            jax.ShapeDtypeStruct((n, d), jnp.float32),
        ),
    )(deg3, x, w1)


def _tc_layer2(s1, g1, dinv, b1, w2, n):
    d = g1.shape[1]

    def body(s_ref, g_ref, dinv_ref, b_ref, w_ref, g2_ref):
        s = s_ref[0, :n, :] + s_ref[1, :n, :] + g_ref[...]
        z = jnp.maximum(dinv_ref[...] * s + b_ref[...], 0.0)
        h = jnp.dot(z, w_ref[...], preferred_element_type=jnp.float32,
                    precision=lax.Precision.HIGHEST)
        g2_ref[...] = h * dinv_ref[...]

    return pl.pallas_call(
        body,
        out_shape=jax.ShapeDtypeStruct((n, d), jnp.float32),
    )(s1, g1, dinv, b1, w2)


def _tc_layer3(s2, g2, dinv, b2, n):
    d = g2.shape[1]

    def body(s_ref, g_ref, dinv_ref, b_ref, h_ref):
        s = s_ref[0, :n, :] + s_ref[1, :n, :] + g_ref[...]
        h_ref[...] = dinv_ref[...] * s + b_ref[...]

    return pl.pallas_call(
        body,
        out_shape=jax.ShapeDtypeStruct((n, d), jnp.float32),
    )(s2, g2, dinv, b2)


def kernel(x, edge_index, W1, b1, W2, b2):
    n, d = x.shape
    e = edge_index.shape[1]
    grain = 16 * 16 * CH
    e_pad = -(-e // grain) * grain
    ncht = e_pad // CH
    n_acc = -(-(n + 1) // CH) * CH
    stripe = n_acc // NS
    scat_split = _core_split(ncht, SPLIT_SCAT, 16)
    dot_split = _core_split(ncht, SPLIT_DOT, 8)
    extra = abs(dot_split[0] - dot_split[1])
    pad = e_pad - e

    src = edge_index[0]
    dst = edge_index[1]
    srcp2 = jnp.concatenate(
        [src, jnp.zeros((pad + extra * CH,), jnp.int32)]).reshape(ncht + extra, CH)
    dstp2_sc = jnp.concatenate(
        [dst, jnp.full((pad,), n, jnp.int32), jnp.zeros((extra * CH,), jnp.int32)]
    ).reshape(ncht + extra, CH)
    dstp2_g = jnp.concatenate(
        [dst, jnp.zeros((pad + extra * CH,), jnp.int32)]).reshape(ncht + extra, CH)
    ones = jnp.ones((CH,), jnp.float32)
    zrow = jnp.zeros((stripe,), jnp.float32)
    zblk = jnp.zeros((stripe, d), jnp.float32)

    deg = _sc_deg(dstp2_sc, ones, zrow, n_acc, ncht)
    dinv, g1 = _tc_layer1(deg.reshape(NC, n_acc, 1), x, W1, n)
    s1 = _sc_scatter(g1, srcp2, dstp2_sc, zblk, n_acc, ncht, scat_split)
    g2 = _tc_layer2(s1, g1, dinv, b1.reshape(1, d), W2, n)
    s2 = _sc_scatter(g2, srcp2, dstp2_sc, zblk, n_acc, ncht, scat_split)
    h2 = _tc_layer3(s2, g2, dinv, b2.reshape(1, d), n)
    logits = _sc_edge_dot(h2, srcp2, dstp2_g, ncht, e_pad, dot_split)
    return logits[:e]


# speedup vs baseline: 1.9622x; 1.9622x over previous
import functools

import jax
import jax.numpy as jnp
from jax import lax
from jax.experimental import pallas as pl
from jax.experimental.pallas import tpu as pltpu
from jax.experimental.pallas import tpu_sc as plsc

NC = 2
NS = 16
NW = NC * NS
CH = 128
NB = 4
NB2 = 2
LANES = 16
SPLIT_SCAT = 0.5
SPLIT_DOT = 0.5

_MESH = plsc.VectorSubcoreMesh(
    core_axis_name="c", subcore_axis_name="s", num_cores=NC, num_subcores=NS
)


def _core_split(ncht, f0, align):
    per_core_units = ncht // (16 * align)
    u0 = min(max(int(round(f0 * per_core_units)), 1), per_core_units - 1)
    return u0 * align, (per_core_units - u0) * align


def _sc_deg(dstp2, ones, zrow, n_acc, ncht):
    nch = ncht // NW
    stripe = n_acc // NS

    def body(dstp_hbm, ones_hbm, z_hbm, out_hbm, didx, ones_v, stage_v, deg_sh, sem):
        cid = lax.axis_index("c")
        sid = lax.axis_index("s")
        wid = sid * NC + cid
        pltpu.sync_copy(z_hbm, stage_v)
        pltpu.sync_copy(stage_v, deg_sh.at[pl.ds(sid * stripe, stripe)])
        pltpu.sync_copy(dstp_hbm.at[pl.ds(wid * nch, nch)], didx)
        pltpu.sync_copy(ones_hbm, ones_v)
        plsc.subcore_barrier()

        def fire(i, c):
            pltpu.async_copy(ones_v, deg_sh.at[didx.at[i]], sem, add=True)
            return c

        lax.fori_loop(0, nch, fire, 0)

        def drain(i, c):
            pltpu.make_async_copy(ones_v, deg_sh.at[didx.at[i]], sem).wait()
            return c

        lax.fori_loop(0, nch, drain, 0)
        plsc.subcore_barrier()
        pltpu.sync_copy(deg_sh.at[pl.ds(sid * stripe, stripe)], stage_v)
        pltpu.sync_copy(
            stage_v,
            out_hbm.at[pl.ds(cid * n_acc + sid * stripe, stripe)],
        )

    return pl.kernel(
        body,
        out_type=jax.ShapeDtypeStruct((NC * n_acc,), jnp.float32),
        mesh=_MESH,
        scratch_types=[
            pltpu.VMEM((nch, CH), jnp.int32),
            pltpu.VMEM((CH,), jnp.float32),
            pltpu.VMEM((stripe,), jnp.float32),
            pltpu.VMEM_SHARED((n_acc,), jnp.float32),
            pltpu.SemaphoreType.DMA,
        ],
    )(dstp2, ones, zrow)


@functools.lru_cache(maxsize=None)
def _sc_scatter_kernel(n, d, n_acc, ncht, nch0, nch1):
    SB = 8
    stripe = n_acc // NS

    def body(g_hbm, srcp_hbm, dstp_hbm, z_hbm, out_hbm, sidx, didx, rows, acc_sh, *sems):
        gsems, ssems, isems_s, isems_d = sems[:2], sems[2:4], sems[4:6], sems[6:8]
        cid = lax.axis_index("c")
        sid = lax.axis_index("s")
        nch_w = jnp.where(cid == 0, nch0, nch1)
        cb = pl.multiple_of(jnp.where(cid == 0, sid * nch0, 16 * nch0 + sid * nch1), 8)
        nsc_w = nch_w // SB
        npair_w = nsc_w // 2
        pltpu.sync_copy(z_hbm, acc_sh.at[pl.ds(sid * stripe, stripe)])
        for sl in range(2):
            pltpu.async_copy(srcp_hbm.at[pl.ds(cb + sl * SB, SB)], sidx.at[sl], isems_s[sl])
            pltpu.async_copy(dstp_hbm.at[pl.ds(cb + sl * SB, SB)], didx.at[sl], isems_d[sl])
        pltpu.make_async_copy(srcp_hbm.at[pl.ds(cb, SB)], sidx.at[0], isems_s[0]).wait()
        pltpu.make_async_copy(dstp_hbm.at[pl.ds(cb, SB)], didx.at[0], isems_d[0]).wait()
        pltpu.async_copy(g_hbm.at[sidx.at[0, 0]], rows.at[pl.ds(0, CH)], gsems[0])
        plsc.subcore_barrier()

        def chunk(s0, ph, b, first):
            rs = b % 2
            cur = rows.at[pl.ds(rs * CH, CH)]
            nxt = rows.at[pl.ds((1 - rs) * CH, CH)]
            pltpu.make_async_copy(g_hbm.at[sidx.at[ph, b]], cur, gsems[rs]).wait()
            pltpu.async_copy(cur, acc_sh.at[didx.at[ph, b]], ssems[rs], add=True)
            if not first:
                pidx = didx.at[ph, b - 1] if b > 0 else didx.at[1 - ph, SB - 1]
                pltpu.make_async_copy(nxt, acc_sh.at[pidx], ssems[1 - rs]).wait()
                if b == 0:
                    sr = cb + lax.rem(s0 + 1, nsc_w) * SB
                    pltpu.async_copy(
                        srcp_hbm.at[pl.ds(sr, SB)], sidx.at[1 - ph], isems_s[1 - ph])
                    pltpu.async_copy(
                        dstp_hbm.at[pl.ds(sr, SB)], didx.at[1 - ph], isems_d[1 - ph])
            if b == SB - 1:
                srn = cb + lax.rem(s0 + 1, nsc_w) * SB
                pltpu.make_async_copy(
                    srcp_hbm.at[pl.ds(srn, SB)], sidx.at[1 - ph], isems_s[1 - ph]).wait()
                pltpu.make_async_copy(
                    dstp_hbm.at[pl.ds(srn, SB)], didx.at[1 - ph], isems_d[1 - ph]).wait()
                nref = sidx.at[1 - ph, 0]
            else:
                nref = sidx.at[ph, b + 1]
            pltpu.async_copy(g_hbm.at[nref], nxt, gsems[1 - rs])

        for ph in range(2):
            for b in range(SB):
                chunk(ph, ph, b, ph == 0 and b == 0)

        def pair(p, carry):
            for ph in range(2):
                for b in range(SB):
                    chunk(p * 2 + ph, ph, b, False)
            return carry

        lax.fori_loop(1, npair_w, pair, 0)
        pltpu.make_async_copy(
            rows.at[pl.ds(CH, CH)], acc_sh.at[didx.at[1, SB - 1]], ssems[1]).wait()
        pltpu.make_async_copy(
            g_hbm.at[sidx.at[0, 0]], rows.at[pl.ds(0, CH)], gsems[0]).wait()
        plsc.subcore_barrier()
        pltpu.sync_copy(
            acc_sh.at[pl.ds(sid * stripe, stripe)],
            out_hbm.at[cid, pl.ds(sid * stripe, stripe)],
        )

    return pl.kernel(
        body,
        out_type=jax.ShapeDtypeStruct((NC, n_acc, d), jnp.float32),
        mesh=_MESH,
        scratch_types=[
            pltpu.VMEM((2, SB, CH), jnp.int32),
            pltpu.VMEM((2, SB, CH), jnp.int32),
            pltpu.VMEM((2 * CH, d), jnp.float32),
            pltpu.VMEM_SHARED((n_acc, d), jnp.float32),
        ] + [pltpu.SemaphoreType.DMA] * 8,
    )


def _sc_scatter(g, srcp2, dstp2, zblk, n_acc, ncht, split):
    n, d = g.shape
    return _sc_scatter_kernel(n, d, n_acc, ncht, *split)(g, srcp2, dstp2, zblk)


def _sc_edge_dot(h2p, srcp2, dstp2, ncht, e_pad, split):
    n_acc, d = h2p.shape
    nch0, nch1 = split
    SB = 4
    SCH = CH // 2
    nv = d // LANES
    stripe = n_acc // NS

    def body(h_hbm, s_hbm, t_hbm, out_hbm, sidx, didx, bs, bd, dots, tab_sh, *sems):
        gs, gd, isems_s, isems_d, osems = (
            sems[0:2], sems[2:4], sems[4:6], sems[6:8], sems[8:10])
        cid = lax.axis_index("c")
        sid = lax.axis_index("s")
        nch_w = jnp.where(cid == 0, nch0, nch1)
        cb = pl.multiple_of(jnp.where(cid == 0, sid * nch0, 16 * nch0 + sid * nch1), 8)
        nsc_w = nch_w // SB
        npair_w = nsc_w // 2
        obase = cb * CH
        nsub_w = nch_w * 2
        pltpu.sync_copy(h_hbm.at[pl.ds(sid * stripe, stripe)],
                        tab_sh.at[pl.ds(sid * stripe, stripe)])
        for sl in range(2):
            pltpu.async_copy(s_hbm.at[pl.ds(cb + sl * SB, SB)], sidx.at[sl], isems_s[sl])
            pltpu.async_copy(t_hbm.at[pl.ds(cb + sl * SB, SB)], didx.at[sl], isems_d[sl])
        plsc.subcore_barrier()
        pltpu.make_async_copy(s_hbm.at[pl.ds(cb, SB)], sidx.at[0], isems_s[0]).wait()
        pltpu.make_async_copy(t_hbm.at[pl.ds(cb, SB)], didx.at[0], isems_d[0]).wait()
        pltpu.async_copy(tab_sh.at[sidx.at[0, 0, pl.ds(0, SCH)]], bs.at[pl.ds(0, SCH)], gs[0])
        pltpu.async_copy(tab_sh.at[didx.at[0, 0, pl.ds(0, SCH)]], bd.at[pl.ds(0, SCH)], gd[0])
        lane = lax.iota(jnp.int32, LANES)

        def sub(s0, ph, b, h):
            sl = h
            ls = (s0 * SB + b) * 2 + h
            cur_s = bs.at[pl.ds(sl * SCH, SCH)]
            cur_d = bd.at[pl.ds(sl * SCH, SCH)]
            ir_s = sidx.at[ph, b, pl.ds(h * SCH, SCH)]
            ir_d = didx.at[ph, b, pl.ds(h * SCH, SCH)]
            pltpu.make_async_copy(tab_sh.at[ir_s], cur_s, gs[sl]).wait()
            pltpu.make_async_copy(tab_sh.at[ir_d], cur_d, gd[sl]).wait()
            if b == 0 and h == 0:
                @pl.when(s0 >= 1)
                def _():
                    sr = cb + lax.rem(s0 + 1, nsc_w) * SB
                    pltpu.async_copy(s_hbm.at[pl.ds(sr, SB)], sidx.at[1 - ph], isems_s[1 - ph])
                    pltpu.async_copy(t_hbm.at[pl.ds(sr, SB)], didx.at[1 - ph], isems_d[1 - ph])
            if b == SB - 1 and h == 1:
                srn = cb + lax.rem(s0 + 1, nsc_w) * SB
                pltpu.make_async_copy(
                    s_hbm.at[pl.ds(srn, SB)], sidx.at[1 - ph], isems_s[1 - ph]).wait()
                pltpu.make_async_copy(
                    t_hbm.at[pl.ds(srn, SB)], didx.at[1 - ph], isems_d[1 - ph]).wait()
                nr_s = sidx.at[1 - ph, 0, pl.ds(0, SCH)]
                nr_d = didx.at[1 - ph, 0, pl.ds(0, SCH)]
            elif h == 1:
                nr_s = sidx.at[ph, b + 1, pl.ds(0, SCH)]
                nr_d = didx.at[ph, b + 1, pl.ds(0, SCH)]
            else:
                nr_s = sidx.at[ph, b, pl.ds(SCH, SCH)]
                nr_d = didx.at[ph, b, pl.ds(SCH, SCH)]
            pltpu.async_copy(tab_sh.at[nr_s], bs.at[pl.ds((1 - sl) * SCH, SCH)], gs[1 - sl])
            pltpu.async_copy(tab_sh.at[nr_d], bd.at[pl.ds((1 - sl) * SCH, SCH)], gd[1 - sl])

            @pl.when(ls >= 2)
            def _():
                pltpu.make_async_copy(
                    dots.at[pl.ds(sl * SCH, SCH)],
                    out_hbm.at[pl.ds(obase + (ls - 2) * SCH, SCH)],
                    osems[sl],
                ).wait()
            def group(gi, c2):
                def edot(k, vec):
                    e = sl * SCH + gi * LANES + k
                    v = bs[e, pl.ds(0, LANES)] * bd[e, pl.ds(0, LANES)]
                    for jj in range(1, nv):
                        v = v + bs[e, pl.ds(jj * LANES, LANES)] * bd[e, pl.ds(jj * LANES, LANES)]
                    for sh in (8, 4, 2, 1):
                        v = v + jnp.take_along_axis(
                            v, lane ^ sh, axis=0, mode="promise_in_bounds")
                    return jnp.where(lane == k, v, vec)

                dots[pl.ds(sl * SCH + gi * LANES, LANES)] = lax.fori_loop(
                    0, LANES, edot, jnp.zeros((LANES,), jnp.float32))
                return c2

            lax.fori_loop(0, SCH // LANES, group, 0)
            pltpu.async_copy(
                dots.at[pl.ds(sl * SCH, SCH)],
                out_hbm.at[pl.ds(obase + ls * SCH, SCH)],
                osems[sl],
            )

        def pair(p, carry):
            for ph in range(2):
                for b in range(SB):
                    for h in range(2):
                        sub(p * 2 + ph, ph, b, h)
            return carry

        lax.fori_loop(0, npair_w, pair, 0)
        for sl in range(2):
            pltpu.make_async_copy(
                dots.at[pl.ds(sl * SCH, SCH)],
                out_hbm.at[pl.ds(obase + (nsub_w - 2 + sl) * SCH, SCH)],
                osems[sl],
            ).wait()
        pltpu.make_async_copy(
            tab_sh.at[sidx.at[0, 0, pl.ds(0, SCH)]], bs.at[pl.ds(0, SCH)], gs[0]).wait()
        pltpu.make_async_copy(
            tab_sh.at[didx.at[0, 0, pl.ds(0, SCH)]], bd.at[pl.ds(0, SCH)], gd[0]).wait()

    return pl.kernel(
        body,
        out_type=jax.ShapeDtypeStruct((e_pad,), jnp.float32),
        mesh=_MESH,
        scratch_types=[
            pltpu.VMEM((2, SB, CH), jnp.int32),
            pltpu.VMEM((2, SB, CH), jnp.int32),
            pltpu.VMEM((2 * SCH, d), jnp.float32),
            pltpu.VMEM((2 * SCH, d), jnp.float32),
            pltpu.VMEM((2 * SCH,), jnp.float32),
            pltpu.VMEM_SHARED((n_acc, d), jnp.float32),
        ] + [pltpu.SemaphoreType.DMA] * 10,
    )(h2p, srcp2, dstp2)


def _tc_layer1(deg3, x, w1, n):
    d = x.shape[1]

    def body(deg_ref, x_ref, w_ref, dinv_ref, g_ref):
        dv = lax.rsqrt(deg_ref[0] + deg_ref[1] + 1.0)[:n]
        h = jnp.dot(x_ref[...], w_ref[...], preferred_element_type=jnp.float32,
                    precision=lax.Precision.HIGHEST)
        dinv_ref[...] = dv
        g_ref[...] = h * dv

    return pl.pallas_call(
        body,
        out_shape=(
            jax.ShapeDtypeStruct((n, 1), jnp.float32),
            jax.ShapeDtypeStruct((n, d), jnp.float32),
        ),
    )(deg3, x, w1)


def _tc_layer2(s1, g1, dinv, b1, w2, n):
    d = g1.shape[1]

    def body(s_ref, g_ref, dinv_ref, b_ref, w_ref, g2_ref):
        s = s_ref[0, :n, :] + s_ref[1, :n, :] + g_ref[...]
        z = jnp.maximum(dinv_ref[...] * s + b_ref[...], 0.0)
        h = jnp.dot(z, w_ref[...], preferred_element_type=jnp.float32,
                    precision=lax.Precision.HIGHEST)
        g2_ref[...] = h * dinv_ref[...]

    return pl.pallas_call(
        body,
        out_shape=jax.ShapeDtypeStruct((n, d), jnp.float32),
    )(s1, g1, dinv, b1, w2)


def _tc_layer3(s2, g2, dinv, b2, n):
    d = g2.shape[1]

    def body(s_ref, g_ref, dinv_ref, b_ref, h_ref):
        s = s_ref[0, :n, :] + s_ref[1, :n, :] + g_ref[...]
        h_ref[...] = dinv_ref[...] * s + b_ref[...]

    return pl.pallas_call(
        body,
        out_shape=jax.ShapeDtypeStruct((n, d), jnp.float32),
    )(s2, g2, dinv, b2)


def kernel(x, edge_index, W1, b1, W2, b2):
    n, d = x.shape
    e = edge_index.shape[1]
    grain = 16 * 16 * CH
    e_pad = -(-e // grain) * grain
    ncht = e_pad // CH
    n_acc = -(-(n + 1) // CH) * CH
    stripe = n_acc // NS
    scat_split = _core_split(ncht, SPLIT_SCAT, 16)
    dot_split = _core_split(ncht, SPLIT_DOT, 16)
    extra = 0
    pad = e_pad - e

    src = edge_index[0]
    dst = edge_index[1]
    srcp2 = jnp.concatenate(
        [src, jnp.zeros((pad + extra * CH,), jnp.int32)]).reshape(ncht + extra, CH)
    dstp2_sc = jnp.concatenate(
        [dst, jnp.full((pad,), n, jnp.int32), jnp.zeros((extra * CH,), jnp.int32)]
    ).reshape(ncht + extra, CH)
    dstp2_g = jnp.concatenate(
        [dst, jnp.zeros((pad + extra * CH,), jnp.int32)]).reshape(ncht + extra, CH)
    ones = jnp.ones((CH,), jnp.float32)
    zrow = jnp.zeros((stripe,), jnp.float32)
    zblk = jnp.zeros((stripe, d), jnp.float32)

    deg = _sc_deg(dstp2_sc, ones, zrow, n_acc, ncht)
    dinv, g1 = _tc_layer1(deg.reshape(NC, n_acc, 1), x, W1, n)
    s1 = _sc_scatter(g1, srcp2, dstp2_sc, zblk, n_acc, ncht, scat_split)
    g2 = _tc_layer2(s1, g1, dinv, b1.reshape(1, d), W2, n)
    s2 = _sc_scatter(g2, srcp2, dstp2_sc, zblk, n_acc, ncht, scat_split)
    h2 = _tc_layer3(s2, g2, dinv, b2.reshape(1, d), n)
    h2p = jnp.concatenate([h2, jnp.zeros((n_acc - n, d), jnp.float32)])
    logits = _sc_edge_dot(h2p, srcp2, dstp2_g, ncht, e_pad, dot_split)
    return logits[:e]


# speedup vs baseline: 1.9626x; 1.0002x over previous
import functools

import jax
import jax.numpy as jnp
from jax import lax
from jax.experimental import pallas as pl
from jax.experimental.pallas import tpu as pltpu
from jax.experimental.pallas import tpu_sc as plsc

NC = 2
NS = 16
NW = NC * NS
CH = 128
LANES = 16
SPLIT_SCAT = 0.5
SPLIT_DOT = 0.5

_MESH = plsc.VectorSubcoreMesh(
    core_axis_name="c", subcore_axis_name="s", num_cores=NC, num_subcores=NS
)


def _core_split(ncht, f0, align):
    per_core_units = ncht // (16 * align)
    u0 = min(max(int(round(f0 * per_core_units)), 1), per_core_units - 1)
    return u0 * align, (per_core_units - u0) * align


def _sc_deg(dstp2, ones, zrow, n_acc, ncht):
    nch = ncht // NW
    stripe = n_acc // NS

    def body(dstp_hbm, ones_hbm, z_hbm, out_hbm, didx, ones_v, stage_v, deg_sh, sem):
        cid = lax.axis_index("c")
        sid = lax.axis_index("s")
        wid = sid * NC + cid
        pltpu.sync_copy(z_hbm, stage_v)
        pltpu.sync_copy(stage_v, deg_sh.at[pl.ds(sid * stripe, stripe)])
        pltpu.sync_copy(dstp_hbm.at[pl.ds(wid * nch, nch)], didx)
        pltpu.sync_copy(ones_hbm, ones_v)
        plsc.subcore_barrier()

        def fire(i, c):
            pltpu.async_copy(ones_v, deg_sh.at[didx.at[i]], sem, add=True)
            return c

        lax.fori_loop(0, nch, fire, 0)

        def drain(i, c):
            pltpu.make_async_copy(ones_v, deg_sh.at[didx.at[i]], sem).wait()
            return c

        lax.fori_loop(0, nch, drain, 0)
        plsc.subcore_barrier()
        pltpu.sync_copy(deg_sh.at[pl.ds(sid * stripe, stripe)], stage_v)
        pltpu.sync_copy(
            stage_v,
            out_hbm.at[pl.ds(cid * n_acc + sid * stripe, stripe)],
        )

    return pl.kernel(
        body,
        out_type=jax.ShapeDtypeStruct((NC * n_acc,), jnp.float32),
        mesh=_MESH,
        scratch_types=[
            pltpu.VMEM((nch, CH), jnp.int32),
            pltpu.VMEM((CH,), jnp.float32),
            pltpu.VMEM((stripe,), jnp.float32),
            pltpu.VMEM_SHARED((n_acc,), jnp.float32),
            pltpu.SemaphoreType.DMA,
        ],
    )(dstp2, ones, zrow)


@functools.lru_cache(maxsize=None)
def _sc_scatter_kernel(n, d, n_acc, ncht, nch0, nch1):
    SB = 8
    stripe = n_acc // NS

    def body(g_hbm, srcp_hbm, dstp_hbm, z_hbm, out_hbm, sidx, didx, rows, acc_sh, *sems):
        gsems, ssems, isems_s, isems_d = sems[:2], sems[2:4], sems[4:6], sems[6:8]
        cid = lax.axis_index("c")
        sid = lax.axis_index("s")
        nch_w = jnp.where(cid == 0, nch0, nch1)
        cb = pl.multiple_of(jnp.where(cid == 0, sid * nch0, 16 * nch0 + sid * nch1), 8)
        nsc_w = nch_w // SB
        npair_w = nsc_w // 2
        pltpu.sync_copy(z_hbm, acc_sh.at[pl.ds(sid * stripe, stripe)])
        for sl in range(2):
            pltpu.async_copy(srcp_hbm.at[pl.ds(cb + sl * SB, SB)], sidx.at[sl], isems_s[sl])
            pltpu.async_copy(dstp_hbm.at[pl.ds(cb + sl * SB, SB)], didx.at[sl], isems_d[sl])
        pltpu.make_async_copy(srcp_hbm.at[pl.ds(cb, SB)], sidx.at[0], isems_s[0]).wait()
        pltpu.make_async_copy(dstp_hbm.at[pl.ds(cb, SB)], didx.at[0], isems_d[0]).wait()
        pltpu.async_copy(g_hbm.at[sidx.at[0, 0]], rows.at[pl.ds(0, CH)], gsems[0])
        plsc.subcore_barrier()

        def chunk(s0, ph, b, first):
            rs = b % 2
            cur = rows.at[pl.ds(rs * CH, CH)]
            nxt = rows.at[pl.ds((1 - rs) * CH, CH)]
            pltpu.make_async_copy(g_hbm.at[sidx.at[ph, b]], cur, gsems[rs]).wait()
            pltpu.async_copy(cur, acc_sh.at[didx.at[ph, b]], ssems[rs], add=True)
            if not first:
                pidx = didx.at[ph, b - 1] if b > 0 else didx.at[1 - ph, SB - 1]
                pltpu.make_async_copy(nxt, acc_sh.at[pidx], ssems[1 - rs]).wait()
                if b == 0:
                    sr = cb + lax.rem(s0 + 1, nsc_w) * SB
                    pltpu.async_copy(
                        srcp_hbm.at[pl.ds(sr, SB)], sidx.at[1 - ph], isems_s[1 - ph])
                    pltpu.async_copy(
                        dstp_hbm.at[pl.ds(sr, SB)], didx.at[1 - ph], isems_d[1 - ph])
            if b == SB - 1:
                srn = cb + lax.rem(s0 + 1, nsc_w) * SB
                pltpu.make_async_copy(
                    srcp_hbm.at[pl.ds(srn, SB)], sidx.at[1 - ph], isems_s[1 - ph]).wait()
                pltpu.make_async_copy(
                    dstp_hbm.at[pl.ds(srn, SB)], didx.at[1 - ph], isems_d[1 - ph]).wait()
                nref = sidx.at[1 - ph, 0]
            else:
                nref = sidx.at[ph, b + 1]
            pltpu.async_copy(g_hbm.at[nref], nxt, gsems[1 - rs])

        for ph in range(2):
            for b in range(SB):
                chunk(ph, ph, b, ph == 0 and b == 0)

        def pair(p, carry):
            for ph in range(2):
                for b in range(SB):
                    chunk(p * 2 + ph, ph, b, False)
            return carry

        lax.fori_loop(1, npair_w, pair, 0)
        pltpu.make_async_copy(
            rows.at[pl.ds(CH, CH)], acc_sh.at[didx.at[1, SB - 1]], ssems[1]).wait()
        pltpu.make_async_copy(
            g_hbm.at[sidx.at[0, 0]], rows.at[pl.ds(0, CH)], gsems[0]).wait()
        plsc.subcore_barrier()
        pltpu.sync_copy(
            acc_sh.at[pl.ds(sid * stripe, stripe)],
            out_hbm.at[cid, pl.ds(sid * stripe, stripe)],
        )

    return pl.kernel(
        body,
        out_type=jax.ShapeDtypeStruct((NC, n_acc, d), jnp.float32),
        mesh=_MESH,
        scratch_types=[
            pltpu.VMEM((2, SB, CH), jnp.int32),
            pltpu.VMEM((2, SB, CH), jnp.int32),
            pltpu.VMEM((2 * CH, d), jnp.float32),
            pltpu.VMEM_SHARED((n_acc, d), jnp.float32),
        ] + [pltpu.SemaphoreType.DMA] * 8,
    )


def _sc_scatter(g, srcp2, dstp2, zblk, n_acc, ncht, split):
    n, d = g.shape
    return _sc_scatter_kernel(n, d, n_acc, ncht, *split)(g, srcp2, dstp2, zblk)


def _sc_edge_dot(h2p, srcp2, dstp2, ncht, e_pad, split):
    n_acc, d = h2p.shape
    nch0, nch1 = split
    SB = 4
    SCH = CH // 2
    nv = d // LANES
    stripe = n_acc // NS

    def body(h_hbm, s_hbm, t_hbm, out_hbm, sidx, didx, bs, bd, dots, tab_sh, *sems):
        gs, gd, isems_s, isems_d, osems = (
            sems[0:2], sems[2:4], sems[4:6], sems[6:8], sems[8:10])
        cid = lax.axis_index("c")
        sid = lax.axis_index("s")
        nch_w = jnp.where(cid == 0, nch0, nch1)
        cb = pl.multiple_of(jnp.where(cid == 0, sid * nch0, 16 * nch0 + sid * nch1), 8)
        nsc_w = nch_w // SB
        npair_w = nsc_w // 2
        obase = cb * CH
        nsub_w = nch_w * 2
        pltpu.sync_copy(h_hbm.at[pl.ds(sid * stripe, stripe)],
                        tab_sh.at[pl.ds(sid * stripe, stripe)])
        for sl in range(2):
            pltpu.async_copy(s_hbm.at[pl.ds(cb + sl * SB, SB)], sidx.at[sl], isems_s[sl])
            pltpu.async_copy(t_hbm.at[pl.ds(cb + sl * SB, SB)], didx.at[sl], isems_d[sl])
        plsc.subcore_barrier()
        pltpu.make_async_copy(s_hbm.at[pl.ds(cb, SB)], sidx.at[0], isems_s[0]).wait()
        pltpu.make_async_copy(t_hbm.at[pl.ds(cb, SB)], didx.at[0], isems_d[0]).wait()
        pltpu.async_copy(tab_sh.at[sidx.at[0, 0, pl.ds(0, SCH)]], bs.at[pl.ds(0, SCH)], gs[0])
        pltpu.async_copy(tab_sh.at[didx.at[0, 0, pl.ds(0, SCH)]], bd.at[pl.ds(0, SCH)], gd[0])
        lane = lax.iota(jnp.int32, LANES)

        def sub(s0, ph, b, h):
            sl = h
            ls = (s0 * SB + b) * 2 + h
            cur_s = bs.at[pl.ds(sl * SCH, SCH)]
            cur_d = bd.at[pl.ds(sl * SCH, SCH)]
            ir_s = sidx.at[ph, b, pl.ds(h * SCH, SCH)]
            ir_d = didx.at[ph, b, pl.ds(h * SCH, SCH)]
            pltpu.make_async_copy(tab_sh.at[ir_s], cur_s, gs[sl]).wait()
            pltpu.make_async_copy(tab_sh.at[ir_d], cur_d, gd[sl]).wait()
            if b == 0 and h == 0:
                @pl.when(s0 >= 1)
                def _():
                    sr = cb + lax.rem(s0 + 1, nsc_w) * SB
                    pltpu.async_copy(s_hbm.at[pl.ds(sr, SB)], sidx.at[1 - ph], isems_s[1 - ph])
                    pltpu.async_copy(t_hbm.at[pl.ds(sr, SB)], didx.at[1 - ph], isems_d[1 - ph])
            if b == SB - 1 and h == 1:
                srn = cb + lax.rem(s0 + 1, nsc_w) * SB
                pltpu.make_async_copy(
                    s_hbm.at[pl.ds(srn, SB)], sidx.at[1 - ph], isems_s[1 - ph]).wait()
                pltpu.make_async_copy(
                    t_hbm.at[pl.ds(srn, SB)], didx.at[1 - ph], isems_d[1 - ph]).wait()
                nr_s = sidx.at[1 - ph, 0, pl.ds(0, SCH)]
                nr_d = didx.at[1 - ph, 0, pl.ds(0, SCH)]
            elif h == 1:
                nr_s = sidx.at[ph, b + 1, pl.ds(0, SCH)]
                nr_d = didx.at[ph, b + 1, pl.ds(0, SCH)]
            else:
                nr_s = sidx.at[ph, b, pl.ds(SCH, SCH)]
                nr_d = didx.at[ph, b, pl.ds(SCH, SCH)]
            pltpu.async_copy(tab_sh.at[nr_s], bs.at[pl.ds((1 - sl) * SCH, SCH)], gs[1 - sl])
            pltpu.async_copy(tab_sh.at[nr_d], bd.at[pl.ds((1 - sl) * SCH, SCH)], gd[1 - sl])

            @pl.when(ls >= 2)
            def _():
                pltpu.make_async_copy(
                    dots.at[pl.ds(sl * SCH, SCH)],
                    out_hbm.at[pl.ds(obase + (ls - 2) * SCH, SCH)],
                    osems[sl],
                ).wait()
            def group(gi, c2):
                def edot(k, vec):
                    e = sl * SCH + gi * LANES + k
                    v = bs[e, pl.ds(0, LANES)] * bd[e, pl.ds(0, LANES)]
                    for jj in range(1, nv):
                        v = v + bs[e, pl.ds(jj * LANES, LANES)] * bd[e, pl.ds(jj * LANES, LANES)]
                    for sh in (8, 4, 2, 1):
                        v = v + jnp.take_along_axis(
                            v, lane ^ sh, axis=0, mode="promise_in_bounds")
                    return jnp.where(lane == k, v, vec)

                dots[pl.ds(sl * SCH + gi * LANES, LANES)] = lax.fori_loop(
                    0, LANES, edot, jnp.zeros((LANES,), jnp.float32))
                return c2

            lax.fori_loop(0, SCH // LANES, group, 0)
            pltpu.async_copy(
                dots.at[pl.ds(sl * SCH, SCH)],
                out_hbm.at[pl.ds(obase + ls * SCH, SCH)],
                osems[sl],
            )

        def pair(p, carry):
            for ph in range(2):
                for b in range(SB):
                    for h in range(2):
                        sub(p * 2 + ph, ph, b, h)
            return carry

        lax.fori_loop(0, npair_w, pair, 0)
        for sl in range(2):
            pltpu.make_async_copy(
                dots.at[pl.ds(sl * SCH, SCH)],
                out_hbm.at[pl.ds(obase + (nsub_w - 2 + sl) * SCH, SCH)],
                osems[sl],
            ).wait()
        pltpu.make_async_copy(
            tab_sh.at[sidx.at[0, 0, pl.ds(0, SCH)]], bs.at[pl.ds(0, SCH)], gs[0]).wait()
        pltpu.make_async_copy(
            tab_sh.at[didx.at[0, 0, pl.ds(0, SCH)]], bd.at[pl.ds(0, SCH)], gd[0]).wait()

    return pl.kernel(
        body,
        out_type=jax.ShapeDtypeStruct((e_pad,), jnp.float32),
        mesh=_MESH,
        scratch_types=[
            pltpu.VMEM((2, SB, CH), jnp.int32),
            pltpu.VMEM((2, SB, CH), jnp.int32),
            pltpu.VMEM((2 * SCH, d), jnp.float32),
            pltpu.VMEM((2 * SCH, d), jnp.float32),
            pltpu.VMEM((2 * SCH,), jnp.float32),
            pltpu.VMEM_SHARED((n_acc, d), jnp.float32),
        ] + [pltpu.SemaphoreType.DMA] * 10,
    )(h2p, srcp2, dstp2)


def _tc_layer1(deg3, x, w1, n):
    d = x.shape[1]

    def body(deg_ref, x_ref, w_ref, dinv_ref, g_ref):
        dv = lax.rsqrt(deg_ref[0] + deg_ref[1] + 1.0)[:n]
        h = jnp.dot(x_ref[...], w_ref[...], preferred_element_type=jnp.float32,
                    precision=lax.Precision.HIGHEST)
        dinv_ref[...] = dv
        g_ref[...] = h * dv

    return pl.pallas_call(
        body,
        out_shape=(
            jax.ShapeDtypeStruct((n, 1), jnp.float32),
            jax.ShapeDtypeStruct((n, d), jnp.float32),
        ),
    )(deg3, x, w1)


def _tc_layer2(s1, g1, dinv, b1, w2, n):
    d = g1.shape[1]

    def body(s_ref, g_ref, dinv_ref, b_ref, w_ref, g2_ref):
        s = s_ref[0, :n, :] + s_ref[1, :n, :] + g_ref[...]
        z = jnp.maximum(dinv_ref[...] * s + b_ref[...], 0.0)
        h = jnp.dot(z, w_ref[...], preferred_element_type=jnp.float32,
                    precision=lax.Precision.HIGHEST)
        g2_ref[...] = h * dinv_ref[...]

    return pl.pallas_call(
        body,
        out_shape=jax.ShapeDtypeStruct((n, d), jnp.float32),
    )(s1, g1, dinv, b1, w2)


def _tc_layer3(s2, g2, dinv, b2, n):
    d = g2.shape[1]

    def body(s_ref, g_ref, dinv_ref, b_ref, h_ref):
        s = s_ref[0, :n, :] + s_ref[1, :n, :] + g_ref[...]
        h_ref[...] = dinv_ref[...] * s + b_ref[...]

    return pl.pallas_call(
        body,
        out_shape=jax.ShapeDtypeStruct((n, d), jnp.float32),
    )(s2, g2, dinv, b2)


def kernel(x, edge_index, W1, b1, W2, b2):
    n, d = x.shape
    e = edge_index.shape[1]
    grain = 16 * 16 * CH
    e_pad = -(-e // grain) * grain
    ncht = e_pad // CH
    n_acc = -(-(n + 1) // CH) * CH
    stripe = n_acc // NS
    scat_split = _core_split(ncht, SPLIT_SCAT, 16)
    dot_split = _core_split(ncht, SPLIT_DOT, 16)
    extra = 0
    pad = e_pad - e

    src = edge_index[0]
    dst = edge_index[1]
    srcp2 = jnp.concatenate(
        [src, jnp.zeros((pad + extra * CH,), jnp.int32)]).reshape(ncht + extra, CH)
    dstp2_sc = jnp.concatenate(
        [dst, jnp.full((pad,), n, jnp.int32), jnp.zeros((extra * CH,), jnp.int32)]
    ).reshape(ncht + extra, CH)
    dstp2_g = jnp.concatenate(
        [dst, jnp.zeros((pad + extra * CH,), jnp.int32)]).reshape(ncht + extra, CH)
    ones = jnp.ones((CH,), jnp.float32)
    zrow = jnp.zeros((stripe,), jnp.float32)
    zblk = jnp.zeros((stripe, d), jnp.float32)

    deg = _sc_deg(dstp2_sc, ones, zrow, n_acc, ncht)
    dinv, g1 = _tc_layer1(deg.reshape(NC, n_acc, 1), x, W1, n)
    s1 = _sc_scatter(g1, srcp2, dstp2_sc, zblk, n_acc, ncht, scat_split)
    g2 = _tc_layer2(s1, g1, dinv, b1.reshape(1, d), W2, n)
    s2 = _sc_scatter(g2, srcp2, dstp2_sc, zblk, n_acc, ncht, scat_split)
    h2 = _tc_layer3(s2, g2, dinv, b2.reshape(1, d), n)
    h2p = jnp.concatenate([h2, jnp.zeros((n_acc - n, d), jnp.float32)])
    logits = _sc_edge_dot(h2p, srcp2, dstp2_g, ncht, e_pad, dot_split)
    return logits[:e]
